# baseline (speedup 1.0000x reference)
import functools

import jax
import jax.numpy as jnp
from jax import lax
from jax.experimental import pallas as pl
from jax.experimental.pallas import tpu as pltpu
from jax.experimental.pallas import tpu_sc as plsc

B = 16384
D = 32
H = 64
NC, NS = 2, 16
NW = NC * NS
B_PER_W = B // NW
CHUNK = 128
NCHUNK = B_PER_W // CHUNK


def _sc_gather(uids2d, iids2d, user_table, item_table):
    mesh = plsc.VectorSubcoreMesh(core_axis_name="c", subcore_axis_name="s")

    @functools.partial(
        pl.kernel,
        mesh=mesh,
        compiler_params=pltpu.CompilerParams(use_tc_tiling_on_sc=False),
        out_type=[
            jax.ShapeDtypeStruct((B, D), jnp.float32),
            jax.ShapeDtypeStruct((B, D), jnp.float32),
        ],
        scratch_types=[
            pltpu.VMEM((NCHUNK, CHUNK), jnp.int32),
            pltpu.VMEM((NCHUNK, CHUNK), jnp.int32),
            pltpu.VMEM((B_PER_W, D), jnp.float32),
            pltpu.VMEM((B_PER_W, D), jnp.float32),
            pltpu.SemaphoreType.DMA,
            pltpu.SemaphoreType.DMA,
        ],
    )
    def gather_kernel(uids, iids, utab, itab, uout, iout,
                      uidx, iidx, urows, irows, usem, isem):
        wid = lax.axis_index("s") * NC + lax.axis_index("c")
        base = wid * B_PER_W
        row0 = wid * NCHUNK
        pltpu.sync_copy(uids.at[pl.ds(row0, NCHUNK)], uidx)
        pltpu.sync_copy(iids.at[pl.ds(row0, NCHUNK)], iidx)
        copies = []
        for j in range(NCHUNK):
            copies.append(pltpu.async_copy(
                utab.at[uidx.at[j]], urows.at[pl.ds(j * CHUNK, CHUNK)], usem))
            copies.append(pltpu.async_copy(
                itab.at[iidx.at[j]], irows.at[pl.ds(j * CHUNK, CHUNK)], isem))
        for c in copies:
            c.wait()
        pltpu.sync_copy(urows, uout.at[pl.ds(base, B_PER_W)])
        pltpu.sync_copy(irows, iout.at[pl.ds(base, B_PER_W)])

    return gather_kernel(uids2d, iids2d, user_table, item_table)


BLK = 2048


def _mlp_body(u_ref, i_ref, w1u_ref, w1i_ref, b1_ref, w2_ref, b2_ref, o_ref):
    h = (jnp.dot(u_ref[...], w1u_ref[...], preferred_element_type=jnp.float32)
         + jnp.dot(i_ref[...], w1i_ref[...], preferred_element_type=jnp.float32)
         + b1_ref[...])
    h = jnp.maximum(h, 0.0)
    z = jnp.sum(h * w2_ref[...], axis=1, keepdims=True) + b2_ref[...]
    o_ref[...] = jax.nn.sigmoid(z)


def _tc_mlp(u_vec, i_vec, w1u, w1i, b1_2d, w2_2d, b2_2d):
    return pl.pallas_call(
        _mlp_body,
        grid=(B // BLK,),
        in_specs=[
            pl.BlockSpec((BLK, D), lambda i: (i, 0)),
            pl.BlockSpec((BLK, D), lambda i: (i, 0)),
            pl.BlockSpec((D, H), lambda i: (0, 0)),
            pl.BlockSpec((D, H), lambda i: (0, 0)),
            pl.BlockSpec((1, H), lambda i: (0, 0)),
            pl.BlockSpec((1, H), lambda i: (0, 0)),
            pl.BlockSpec((1, 1), lambda i: (0, 0)),
        ],
        out_specs=pl.BlockSpec((BLK, 1), lambda i: (i, 0)),
        out_shape=jax.ShapeDtypeStruct((B, 1), jnp.float32),
    )(u_vec, i_vec, w1u, w1i, b1_2d, w2_2d, b2_2d)


def kernel(user_ids, item_ids, user_table, item_table, W1, b1, W2, b2):
    uids2d = user_ids.astype(jnp.int32).reshape(B // CHUNK, CHUNK)
    iids2d = item_ids.astype(jnp.int32).reshape(B // CHUNK, CHUNK)
    u_vec, i_vec = _sc_gather(uids2d, iids2d, user_table, item_table)
    w1u = W1[:, :D].T
    w1i = W1[:, D:].T
    b1_2d = b1.reshape(1, H)
    w2_2d = W2.reshape(1, H)
    b2_2d = b2.reshape(1, 1)
    return _tc_mlp(u_vec, i_vec, w1u, w1i, b1_2d, w2_2d, b2_2d)
